# baseline (speedup 1.0000x reference)
import functools

import jax
import jax.numpy as jnp
from jax.experimental import pallas as pl
from jax.experimental.pallas import tpu as pltpu

ALPHA = 0.2


GRAPHS_PER_PROGRAM = 2


def _attn_kernel(gs_ref, v_ref, w_ref, t_ref, bias_ref, out_ref):
    g = pl.program_id(0)
    w = w_ref[...]
    t = t_ref[...]
    bias = bias_ref[...]

    wb = w.astype(jnp.bfloat16)
    tb = t.astype(jnp.bfloat16)
    w2b = jax.lax.dot_general(wb, tb, (((1,), (0,)), ((), ())),
                              preferred_element_type=jnp.float32).astype(jnp.bfloat16)
    b2 = jax.lax.dot_general(bias, t, (((1,), (0,)), ((), ())),
                             preferred_element_type=jnp.float32)
    b2t = b2.T

    s_len = v_ref.shape[1]
    col = jax.lax.broadcasted_iota(jnp.int32, (t.shape[1], s_len), 1)
    lane_head = jax.lax.broadcasted_iota(jnp.int32, (t.shape[1], w.shape[1]), 1) // w.shape[0]
    row_head = jax.lax.broadcasted_iota(jnp.int32, (t.shape[1], w.shape[1]), 0)

    for j in range(GRAPHS_PER_PROGRAM):
        gs = gs_ref[g * GRAPHS_PER_PROGRAM + j]
        vb = v_ref[j].astype(jnp.bfloat16)

        at = jax.lax.dot_general(w2b, vb, (((0,), (1,)), ((), ())),
                                 preferred_element_type=jnp.float32) + b2t
        at = jnp.where(at > 0, at, ALPHA * at)

        am = jnp.where(col < gs, at, -jnp.inf)
        m = jnp.max(am, axis=1, keepdims=True)
        ex = jnp.exp(am - m)
        denom = jnp.sum(ex, axis=1, keepdims=True)

        pooled = jax.lax.dot_general(ex.astype(jnp.bfloat16), vb,
                                     (((1,), (0,)), ((), ())),
                                     preferred_element_type=jnp.float32)
        pooled = pooled / denom
        res = jax.lax.dot_general(pooled.astype(jnp.bfloat16), wb,
                                  (((1,), (0,)), ((), ())),
                                  preferred_element_type=jnp.float32)

        out = jnp.sum(jnp.where(lane_head == row_head, res, 0.0), axis=0,
                      keepdims=True)
        out_ref[j] = out + bias


@functools.partial(jax.jit, static_argnames=())
def kernel(V, graph_size, weight, tune_weight, bias):
    b, s, c = V.shape
    n_head = tune_weight.shape[1]
    hc = n_head * c

    flat = tune_weight.reshape(hc)
    kk = jnp.arange(hc, dtype=jnp.int32)
    t_mat = jnp.where((kk[:, None] // c) == jnp.arange(n_head, dtype=jnp.int32)[None, :],
                      flat[:, None], 0.0).astype(jnp.float32)

    gpp = GRAPHS_PER_PROGRAM
    out = pl.pallas_call(
        _attn_kernel,
        grid=(b // gpp,),
        in_specs=[
            pl.BlockSpec(memory_space=pltpu.SMEM),
            pl.BlockSpec((gpp, s, c), lambda i: (i, 0, 0)),
            pl.BlockSpec((c, hc), lambda i: (0, 0)),
            pl.BlockSpec((hc, n_head), lambda i: (0, 0)),
            pl.BlockSpec((1, hc), lambda i: (0, 0)),
        ],
        out_specs=pl.BlockSpec((gpp, 1, hc), lambda i: (i, 0, 0)),
        out_shape=jax.ShapeDtypeStruct((b, 1, hc), jnp.float32),
    )(graph_size, V, weight, t_mat, bias.reshape(1, hc))
    return out.reshape(b, hc)


# speedup vs baseline: 1.0533x; 1.0533x over previous
import functools

import jax
import jax.numpy as jnp
from jax.experimental import pallas as pl
from jax.experimental.pallas import tpu as pltpu

ALPHA = 0.2


GRAPHS_PER_PROGRAM = 4


def _attn_kernel(gs_ref, v_ref, w_ref, t_ref, bias_ref, out_ref):
    g = pl.program_id(0)
    w = w_ref[...]
    t = t_ref[...]
    bias = bias_ref[...]

    wb = w.astype(jnp.bfloat16)
    tb = t.astype(jnp.bfloat16)
    w2b = jax.lax.dot_general(wb, tb, (((1,), (0,)), ((), ())),
                              preferred_element_type=jnp.float32).astype(jnp.bfloat16)
    b2 = jax.lax.dot_general(bias, t, (((1,), (0,)), ((), ())),
                             preferred_element_type=jnp.float32)
    b2t = b2.T

    s_len = v_ref.shape[1]
    col = jax.lax.broadcasted_iota(jnp.int32, (t.shape[1], s_len), 1)
    lane_head = jax.lax.broadcasted_iota(jnp.int32, (t.shape[1], w.shape[1]), 1) // w.shape[0]
    row_head = jax.lax.broadcasted_iota(jnp.int32, (t.shape[1], w.shape[1]), 0)

    for j in range(GRAPHS_PER_PROGRAM):
        gs = gs_ref[g * GRAPHS_PER_PROGRAM + j]
        vb = v_ref[j].astype(jnp.bfloat16)

        at = jax.lax.dot_general(w2b, vb, (((0,), (1,)), ((), ())),
                                 preferred_element_type=jnp.float32) + b2t
        at = jnp.where(at > 0, at, ALPHA * at)

        am = jnp.where(col < gs, at, -jnp.inf)
        m = jnp.max(am, axis=1, keepdims=True)
        ex = jnp.exp(am - m)
        denom = jnp.sum(ex, axis=1, keepdims=True)

        pooled = jax.lax.dot_general(ex.astype(jnp.bfloat16), vb,
                                     (((1,), (0,)), ((), ())),
                                     preferred_element_type=jnp.float32)
        pooled = pooled / denom
        res = jax.lax.dot_general(pooled.astype(jnp.bfloat16), wb,
                                  (((1,), (0,)), ((), ())),
                                  preferred_element_type=jnp.float32)

        out = jnp.sum(jnp.where(lane_head == row_head, res, 0.0), axis=0,
                      keepdims=True)
        out_ref[j] = out + bias


@functools.partial(jax.jit, static_argnames=())
def kernel(V, graph_size, weight, tune_weight, bias):
    b, s, c = V.shape
    n_head = tune_weight.shape[1]
    hc = n_head * c

    flat = tune_weight.reshape(hc)
    kk = jnp.arange(hc, dtype=jnp.int32)
    t_mat = jnp.where((kk[:, None] // c) == jnp.arange(n_head, dtype=jnp.int32)[None, :],
                      flat[:, None], 0.0).astype(jnp.float32)

    gpp = GRAPHS_PER_PROGRAM
    out = pl.pallas_call(
        _attn_kernel,
        grid=(b // gpp,),
        in_specs=[
            pl.BlockSpec(memory_space=pltpu.SMEM),
            pl.BlockSpec((gpp, s, c), lambda i: (i, 0, 0)),
            pl.BlockSpec((c, hc), lambda i: (0, 0)),
            pl.BlockSpec((hc, n_head), lambda i: (0, 0)),
            pl.BlockSpec((1, hc), lambda i: (0, 0)),
        ],
        out_specs=pl.BlockSpec((gpp, 1, hc), lambda i: (i, 0, 0)),
        out_shape=jax.ShapeDtypeStruct((b, 1, hc), jnp.float32),
    )(graph_size, V, weight, t_mat, bias.reshape(1, hc))
    return out.reshape(b, hc)


# speedup vs baseline: 1.0586x; 1.0050x over previous
import functools

import jax
import jax.numpy as jnp
from jax.experimental import pallas as pl
from jax.experimental.pallas import tpu as pltpu

ALPHA = 0.2


GRAPHS_PER_PROGRAM = 4


def _attn_kernel(gs_ref, v_ref, w_ref, t_ref, bias_ref, out_ref):
    g = pl.program_id(0)
    w = w_ref[...]
    t = t_ref[...]
    bias = bias_ref[...]

    wb = w.astype(jnp.bfloat16)
    tb = t.astype(jnp.bfloat16)
    w2b = jax.lax.dot_general(wb, tb, (((1,), (0,)), ((), ())),
                              preferred_element_type=jnp.float32).astype(jnp.bfloat16)
    b2 = jax.lax.dot_general(bias, t, (((1,), (0,)), ((), ())),
                             preferred_element_type=jnp.float32)
    b2t = b2.T

    s_len = v_ref.shape[1]
    col = jax.lax.broadcasted_iota(jnp.int32, (t.shape[1], s_len), 1)
    lane_head = jax.lax.broadcasted_iota(jnp.int32, (t.shape[1], w.shape[1]), 1) // w.shape[0]
    row_head = jax.lax.broadcasted_iota(jnp.int32, (t.shape[1], w.shape[1]), 0)

    for j in range(GRAPHS_PER_PROGRAM):
        gs = gs_ref[g * GRAPHS_PER_PROGRAM + j]
        vb = v_ref[j]

        at = jax.lax.dot_general(w2b.astype(jnp.float32), vb, (((0,), (1,)), ((), ())),
                                 preferred_element_type=jnp.float32) + b2t
        at = jnp.where(at > 0, at, ALPHA * at)

        am = jnp.where(col < gs, at, -jnp.inf)
        m = jnp.max(am, axis=1, keepdims=True)
        ex = jnp.exp(am - m)
        denom = jnp.sum(ex, axis=1, keepdims=True)

        pooled = jax.lax.dot_general(ex, vb,
                                     (((1,), (0,)), ((), ())),
                                     preferred_element_type=jnp.float32)
        pooled = pooled / denom
        res = jax.lax.dot_general(pooled.astype(jnp.bfloat16), wb,
                                  (((1,), (0,)), ((), ())),
                                  preferred_element_type=jnp.float32)

        out = jnp.sum(jnp.where(lane_head == row_head, res, 0.0), axis=0,
                      keepdims=True)
        out_ref[j] = out + bias


@functools.partial(jax.jit, static_argnames=())
def kernel(V, graph_size, weight, tune_weight, bias):
    b, s, c = V.shape
    n_head = tune_weight.shape[1]
    hc = n_head * c

    flat = tune_weight.reshape(hc)
    kk = jnp.arange(hc, dtype=jnp.int32)
    t_mat = jnp.where((kk[:, None] // c) == jnp.arange(n_head, dtype=jnp.int32)[None, :],
                      flat[:, None], 0.0).astype(jnp.float32)

    gpp = GRAPHS_PER_PROGRAM
    out = pl.pallas_call(
        _attn_kernel,
        grid=(b // gpp,),
        in_specs=[
            pl.BlockSpec(memory_space=pltpu.SMEM),
            pl.BlockSpec((gpp, s, c), lambda i: (i, 0, 0)),
            pl.BlockSpec((c, hc), lambda i: (0, 0)),
            pl.BlockSpec((hc, n_head), lambda i: (0, 0)),
            pl.BlockSpec((1, hc), lambda i: (0, 0)),
        ],
        out_specs=pl.BlockSpec((gpp, 1, hc), lambda i: (i, 0, 0)),
        out_shape=jax.ShapeDtypeStruct((b, 1, hc), jnp.float32),
    )(graph_size, V, weight, t_mat, bias.reshape(1, hc))
    return out.reshape(b, hc)


# speedup vs baseline: 1.1906x; 1.1247x over previous
import functools

import jax
import jax.numpy as jnp
from jax.experimental import pallas as pl
from jax.experimental.pallas import tpu as pltpu

ALPHA = 0.2


GRAPHS_PER_PROGRAM = 4


def _attn_kernel(gs_ref, v_ref, w_ref, t_ref, bias_ref, out_ref):
    g = pl.program_id(0)
    w = w_ref[...]
    t = t_ref[...]
    bias = bias_ref[...]

    wb = w.astype(jnp.bfloat16)
    tb = t.astype(jnp.bfloat16)
    w2b = jax.lax.dot_general(wb, tb, (((1,), (0,)), ((), ())),
                              preferred_element_type=jnp.float32).astype(jnp.bfloat16)
    b2 = jax.lax.dot_general(bias, t, (((1,), (0,)), ((), ())),
                             preferred_element_type=jnp.float32)
    b2t = b2.T

    s_len = v_ref.shape[1]
    col = jax.lax.broadcasted_iota(jnp.int32, (t.shape[1], s_len), 1)
    lane_head = jax.lax.broadcasted_iota(jnp.int32, (t.shape[1], w.shape[1]), 1) // w.shape[0]
    row_head = jax.lax.broadcasted_iota(jnp.int32, (t.shape[1], w.shape[1]), 0)

    G = GRAPHS_PER_PROGRAM
    w2f = w2b.astype(jnp.float32)

    def logits(j):
        at = jax.lax.dot_general(w2f, v_ref[j], (((0,), (1,)), ((), ())),
                                 preferred_element_type=jnp.float32) + b2t
        return jnp.where(at > 0, at, ALPHA * at)

    at_j = logits(0)
    for j in range(G):
        at_next = logits(j + 1) if j + 1 < G else None
        gs = gs_ref[g * G + j]
        vb = v_ref[j]

        am = jnp.where(col < gs, at_j, -jnp.inf)
        m = jnp.max(am, axis=1, keepdims=True)
        ex = jnp.exp(am - m)
        denom = jnp.sum(ex, axis=1, keepdims=True)

        pooled = jax.lax.dot_general(ex, vb,
                                     (((1,), (0,)), ((), ())),
                                     preferred_element_type=jnp.float32)
        pooled = pooled / denom
        res = jax.lax.dot_general(pooled.astype(jnp.bfloat16), wb,
                                  (((1,), (0,)), ((), ())),
                                  preferred_element_type=jnp.float32)

        out = jnp.sum(jnp.where(lane_head == row_head, res, 0.0), axis=0,
                      keepdims=True)
        out_ref[j] = out + bias
        at_j = at_next

@functools.partial(jax.jit, static_argnames=())
def kernel(V, graph_size, weight, tune_weight, bias):
    b, s, c = V.shape
    n_head = tune_weight.shape[1]
    hc = n_head * c

    flat = tune_weight.reshape(hc)
    kk = jnp.arange(hc, dtype=jnp.int32)
    t_mat = jnp.where((kk[:, None] // c) == jnp.arange(n_head, dtype=jnp.int32)[None, :],
                      flat[:, None], 0.0).astype(jnp.float32)

    gpp = GRAPHS_PER_PROGRAM
    out = pl.pallas_call(
        _attn_kernel,
        grid=(b // gpp,),
        in_specs=[
            pl.BlockSpec(memory_space=pltpu.SMEM),
            pl.BlockSpec((gpp, s, c), lambda i: (i, 0, 0)),
            pl.BlockSpec((c, hc), lambda i: (0, 0)),
            pl.BlockSpec((hc, n_head), lambda i: (0, 0)),
            pl.BlockSpec((1, hc), lambda i: (0, 0)),
        ],
        out_specs=pl.BlockSpec((gpp, 1, hc), lambda i: (i, 0, 0)),
        out_shape=jax.ShapeDtypeStruct((b, 1, hc), jnp.float32),
    )(graph_size, V, weight, t_mat, bias.reshape(1, hc))
    return out.reshape(b, hc)
